# baseline (speedup 1.0000x reference)
import functools

import jax
import jax.numpy as jnp
from jax import lax
from jax.experimental import pallas as pl
from jax.experimental.pallas import tpu as pltpu
from jax.experimental.pallas import tpu_sc as plsc

E = 320000
N = 10000
D = 128
NC = 2
NS = 16
L = 16
NW = NC * NS
EPW = E // NW
K = 80
NCHUNK = EPW // K
G = K // L
DSTEP = 8
ZROWS_PER_TILE = 1000


def _sc_body(z_hbm, src_hbm, dst_hbm, nrm_hbm, out_hbm,
             zsp, normv, si0, si1, si2, si3, di0, di1, di2, di3,
             rw0, rw1, ob0, ob1,
             isem0, isem1, isem2, isem3,
             g1sem0, g1sem1, g2sem0, g2sem1, osem0, osem1):
    sid = lax.axis_index("s")
    wid = sid * NC + lax.axis_index("c")
    base = wid * EPW

    @pl.when(sid < N // ZROWS_PER_TILE)
    def _():
        pltpu.sync_copy(
            z_hbm.at[pl.ds(sid * ZROWS_PER_TILE, ZROWS_PER_TILE)],
            zsp.at[pl.ds(sid * ZROWS_PER_TILE, ZROWS_PER_TILE)])

    pltpu.sync_copy(nrm_hbm, normv)
    plsc.subcore_barrier()

    sidx = (si0, si1, si2, si3)
    didx = (di0, di1, di2, di3)
    rows = (rw0, rw1)
    obufs = (ob0, ob1)
    isems = (isem0, isem1, isem2, isem3)
    g1sems = (g1sem0, g1sem1)
    g2sems = (g2sem0, g2sem1)
    osems = (osem0, osem1)

    def issue_idx(c, b4):
        pltpu.async_copy(src_hbm.at[pl.ds(base + c * K, K)], sidx[b4],
                         isems[b4])
        pltpu.async_copy(dst_hbm.at[pl.ds(base + c * K, K)], didx[b4],
                         isems[b4])

    def wait_idx(c, b4):
        pltpu.make_async_copy(
            src_hbm.at[pl.ds(base + c * K, K)], sidx[b4], isems[b4]).wait()
        pltpu.make_async_copy(
            dst_hbm.at[pl.ds(base + c * K, K)], didx[b4], isems[b4]).wait()

    def issue_g1(b2, b4):
        pltpu.async_copy(zsp.at[sidx[b4]], rows[b2], g1sems[b2])

    def wait_g1(b2, b4):
        pltpu.make_async_copy(zsp.at[sidx[b4]], rows[b2], g1sems[b2]).wait()

    def issue_g2(b2, b4):
        pltpu.async_copy(zsp.at[didx[b4]], rows[b2], g2sems[b2], add=True)

    def wait_g2(b2, b4):
        pltpu.make_async_copy(zsp.at[didx[b4]], rows[b2], g2sems[b2]).wait()

    def issue_ostore(c, b):
        pltpu.async_copy(obufs[b], out_hbm.at[pl.ds(base + c * K, K)], osems[b])

    def wait_ostore(c, b):
        pltpu.make_async_copy(
            obufs[b], out_hbm.at[pl.ds(base + c * K, K)], osems[b]).wait()

    lanes = lax.iota(jnp.int32, L)

    def compute(b2, b4):
        rref = rows[b2]
        for g in range(G):
            eids = lanes + (g * L)

            def dstep(t, acc):
                for dd in range(DSTEP):
                    dvec = lanes ^ (t * DSTEP + dd)
                    v = plsc.load_gather(rref, [eids, dvec])
                    acc = acc + v * v
                return acc

            acc = lax.fori_loop(0, D // DSTEP, dstep,
                                jnp.zeros((L,), jnp.float32))
            ns = plsc.load_gather(normv, [sidx[b4][pl.ds(g * L, L)]])
            nd = plsc.load_gather(normv, [didx[b4][pl.ds(g * L, L)]])
            dot = (acc - ns - nd) * 0.5
            obufs[b2][pl.ds(g * L, L)] = 1.0 / (1.0 + jnp.exp(-dot))

    def step(c, b2, b4):
        @pl.when(c + 1 < NCHUNK)
        def _():
            wait_g1(1 - b2, (b4 + 1) % 4)
            issue_g2(1 - b2, (b4 + 1) % 4)

        @pl.when(c + 2 < NCHUNK)
        def _():
            wait_idx(c + 2, (b4 + 2) % 4)
        wait_g2(b2, b4)

        @pl.when(c >= 2)
        def _():
            wait_ostore(c - 2, b2)
        compute(b2, b4)
        issue_ostore(c, b2)

        @pl.when(c + 2 < NCHUNK)
        def _():
            issue_g1(b2, (b4 + 2) % 4)

        @pl.when(c + 4 < NCHUNK)
        def _():
            issue_idx(c + 4, b4)

    issue_idx(0, 0)
    issue_idx(1, 1)
    issue_idx(2, 2)
    issue_idx(3, 3)
    wait_idx(0, 0)
    issue_g1(0, 0)
    wait_idx(1, 1)
    wait_g1(0, 0)
    issue_g2(0, 0)
    issue_g1(1, 1)

    def chunk_quad(i, carry):
        for b in range(4):
            step(4 * i + b, b % 2, b)
        return carry

    lax.fori_loop(0, NCHUNK // 4, chunk_quad, 0)
    step(NCHUNK - 1, 0, 0)
    wait_ostore(NCHUNK - 2, 1)
    wait_ostore(NCHUNK - 1, 0)


def _norms_body(z_ref, o_ref):
    z = z_ref[...]
    o_ref[...] = jnp.sum(z * z, axis=1)


@jax.jit
def _run(z, src, dst):
    norms = pl.pallas_call(
        _norms_body,
        out_shape=jax.ShapeDtypeStruct((N,), jnp.float32),
    )(z)

    mesh = plsc.VectorSubcoreMesh(
        core_axis_name="c", subcore_axis_name="s",
        num_cores=NC, num_subcores=NS)
    return pl.kernel(
        _sc_body,
        out_type=jax.ShapeDtypeStruct((E,), jnp.float32),
        mesh=mesh,
        compiler_params=pltpu.CompilerParams(needs_layout_passes=False),
        scratch_types=[
            pltpu.VMEM_SHARED((N, D), jnp.float32),
            pltpu.VMEM((N,), jnp.float32),
            pltpu.VMEM((K,), jnp.int32),
            pltpu.VMEM((K,), jnp.int32),
            pltpu.VMEM((K,), jnp.int32),
            pltpu.VMEM((K,), jnp.int32),
            pltpu.VMEM((K,), jnp.int32),
            pltpu.VMEM((K,), jnp.int32),
            pltpu.VMEM((K,), jnp.int32),
            pltpu.VMEM((K,), jnp.int32),
            pltpu.VMEM((K, D), jnp.float32),
            pltpu.VMEM((K, D), jnp.float32),
            pltpu.VMEM((K,), jnp.float32),
            pltpu.VMEM((K,), jnp.float32),
            pltpu.SemaphoreType.DMA,
            pltpu.SemaphoreType.DMA,
            pltpu.SemaphoreType.DMA,
            pltpu.SemaphoreType.DMA,
            pltpu.SemaphoreType.DMA,
            pltpu.SemaphoreType.DMA,
            pltpu.SemaphoreType.DMA,
            pltpu.SemaphoreType.DMA,
            pltpu.SemaphoreType.DMA,
            pltpu.SemaphoreType.DMA,
        ],
    )(z, src, dst, norms)


def kernel(z, edge_index):
    ei = edge_index.astype(jnp.int32)
    return _run(z, ei[0], ei[1])


# speedup vs baseline: 1.1909x; 1.1909x over previous
import functools

import jax
import jax.numpy as jnp
from jax import lax
from jax.experimental import pallas as pl
from jax.experimental.pallas import tpu as pltpu
from jax.experimental.pallas import tpu_sc as plsc

E = 320000
N = 10000
D = 128
NC = 2
NS = 16
L = 16
NW = NC * NS
EPW = E // NW
K = 80
NCHUNK = EPW // K
G = K // L
DSTEP = 8
ZROWS_PER_TILE = 1000


def _sc_body(z_hbm, src_hbm, dst_hbm, out_hbm,
             si0, si1, di0, di1, sr0, dr0, sr1, dr1, ob0, ob1,
             isem0, isem1, rsem0, rsem1, osem0, osem1):
    sid = lax.axis_index("s")
    wid = sid * NC + lax.axis_index("c")
    base = wid * EPW

    sidx = (si0, si1)
    didx = (di0, di1)
    srows = (sr0, sr1)
    drows = (dr0, dr1)
    obufs = (ob0, ob1)
    isems = (isem0, isem1)
    rsems = (rsem0, rsem1)
    osems = (osem0, osem1)

    def issue_idx(c, b):
        pltpu.async_copy(src_hbm.at[pl.ds(base + c * K, K)], sidx[b], isems[b])
        pltpu.async_copy(dst_hbm.at[pl.ds(base + c * K, K)], didx[b], isems[b])

    def wait_idx(c, b):
        pltpu.make_async_copy(
            src_hbm.at[pl.ds(base + c * K, K)], sidx[b], isems[b]).wait()
        pltpu.make_async_copy(
            dst_hbm.at[pl.ds(base + c * K, K)], didx[b], isems[b]).wait()

    def issue_rows(b):
        pltpu.async_copy(z_hbm.at[sidx[b]], srows[b], rsems[b])
        pltpu.async_copy(z_hbm.at[didx[b]], drows[b], rsems[b])

    def wait_rows(b):
        pltpu.make_async_copy(z_hbm.at[sidx[b]], srows[b], rsems[b]).wait()
        pltpu.make_async_copy(z_hbm.at[didx[b]], drows[b], rsems[b]).wait()

    def issue_ostore(c, b):
        pltpu.async_copy(obufs[b], out_hbm.at[pl.ds(base + c * K, K)], osems[b])

    def wait_ostore(c, b):
        pltpu.make_async_copy(
            obufs[b], out_hbm.at[pl.ds(base + c * K, K)], osems[b]).wait()

    lanes = lax.iota(jnp.int32, L)

    def compute(b):
        sref = srows[b]
        dref = drows[b]
        for g in range(G):
            eids = lanes + (g * L)

            def dstep(t, acc):
                for dd in range(DSTEP):
                    dvec = (lanes + (t * DSTEP + dd)) & (D - 1)
                    sv = plsc.load_gather(sref, [eids, dvec])
                    dv = plsc.load_gather(dref, [eids, dvec])
                    acc = acc + sv * dv
                return acc

            acc = lax.fori_loop(0, D // DSTEP, dstep,
                                jnp.zeros((L,), jnp.float32))
            obufs[b][pl.ds(g * L, L)] = 1.0 / (1.0 + jnp.exp(-acc))

    def step(c, b):
        @pl.when(c + 1 < NCHUNK)
        def _():
            wait_idx(c + 1, 1 - b)
            issue_rows(1 - b)
        wait_rows(b)

        @pl.when(c + 2 < NCHUNK)
        def _():
            issue_idx(c + 2, b)

        @pl.when(c >= 2)
        def _():
            wait_ostore(c - 2, b)
        compute(b)
        issue_ostore(c, b)

    issue_idx(0, 0)
    issue_idx(1, 1)
    wait_idx(0, 0)
    issue_rows(0)

    def chunk_pair(i, carry):
        step(2 * i, 0)
        step(2 * i + 1, 1)
        return carry

    lax.fori_loop(0, NCHUNK // 2, chunk_pair, 0)
    step(NCHUNK - 1, 0)
    wait_ostore(NCHUNK - 2, 1)
    wait_ostore(NCHUNK - 1, 0)


@jax.jit
def _run(z, src, dst):
    mesh = plsc.VectorSubcoreMesh(
        core_axis_name="c", subcore_axis_name="s",
        num_cores=NC, num_subcores=NS)
    return pl.kernel(
        _sc_body,
        out_type=jax.ShapeDtypeStruct((E,), jnp.float32),
        mesh=mesh,
        compiler_params=pltpu.CompilerParams(needs_layout_passes=False),
        scratch_types=[
            pltpu.VMEM((K,), jnp.int32),
            pltpu.VMEM((K,), jnp.int32),
            pltpu.VMEM((K,), jnp.int32),
            pltpu.VMEM((K,), jnp.int32),
            pltpu.VMEM((K, D), jnp.float32),
            pltpu.VMEM((K, D), jnp.float32),
            pltpu.VMEM((K, D), jnp.float32),
            pltpu.VMEM((K, D), jnp.float32),
            pltpu.VMEM((K,), jnp.float32),
            pltpu.VMEM((K,), jnp.float32),
            pltpu.SemaphoreType.DMA,
            pltpu.SemaphoreType.DMA,
            pltpu.SemaphoreType.DMA,
            pltpu.SemaphoreType.DMA,
            pltpu.SemaphoreType.DMA,
            pltpu.SemaphoreType.DMA,
        ],
    )(z, src, dst)


def kernel(z, edge_index):
    ei = edge_index.astype(jnp.int32)
    return _run(z, ei[0], ei[1])


# speedup vs baseline: 1.4409x; 1.2099x over previous
import functools

import jax
import jax.numpy as jnp
from jax import lax
from jax.experimental import pallas as pl
from jax.experimental.pallas import tpu as pltpu
from jax.experimental.pallas import tpu_sc as plsc

E = 320000
N = 10000
D = 128
NC = 2
NS = 16
L = 16
NW = NC * NS
EPW = E // NW
K = 80
NCHUNK = EPW // K
G = K // L
DSTEP = 8
NB = 4


def _sc_body(z_hbm, src_hbm, dst_hbm, out_hbm,
             si0, si1, si2, si3, di0, di1, di2, di3,
             sr0, sr1, sr2, sr3, dr0, dr1, dr2, dr3,
             ob0, ob1, ob2, ob3,
             isem0, isem1, isem2, isem3,
             rsem0, rsem1, rsem2, rsem3,
             osem0, osem1, osem2, osem3):
    wid = lax.axis_index("s") * NC + lax.axis_index("c")
    base = wid * EPW

    sidx = (si0, si1, si2, si3)
    didx = (di0, di1, di2, di3)
    srows = (sr0, sr1, sr2, sr3)
    drows = (dr0, dr1, dr2, dr3)
    obufs = (ob0, ob1, ob2, ob3)
    isems = (isem0, isem1, isem2, isem3)
    rsems = (rsem0, rsem1, rsem2, rsem3)
    osems = (osem0, osem1, osem2, osem3)

    def issue_idx(c, b):
        pltpu.async_copy(src_hbm.at[pl.ds(base + c * K, K)], sidx[b], isems[b])
        pltpu.async_copy(dst_hbm.at[pl.ds(base + c * K, K)], didx[b], isems[b])

    def wait_idx(c, b):
        pltpu.make_async_copy(
            src_hbm.at[pl.ds(base + c * K, K)], sidx[b], isems[b]).wait()
        pltpu.make_async_copy(
            dst_hbm.at[pl.ds(base + c * K, K)], didx[b], isems[b]).wait()

    def issue_rows(b):
        pltpu.async_copy(z_hbm.at[sidx[b]], srows[b], rsems[b])
        pltpu.async_copy(z_hbm.at[didx[b]], drows[b], rsems[b])

    def wait_rows(b):
        pltpu.make_async_copy(z_hbm.at[sidx[b]], srows[b], rsems[b]).wait()
        pltpu.make_async_copy(z_hbm.at[didx[b]], drows[b], rsems[b]).wait()

    def issue_ostore(c, b):
        pltpu.async_copy(obufs[b], out_hbm.at[pl.ds(base + c * K, K)], osems[b])

    def wait_ostore(c, b):
        pltpu.make_async_copy(
            obufs[b], out_hbm.at[pl.ds(base + c * K, K)], osems[b]).wait()

    lanes = lax.iota(jnp.int32, L)

    def compute(b):
        sref = srows[b]
        dref = drows[b]
        for g in range(G):
            eids = lanes + (g * L)

            def dstep(t, acc):
                for dd in range(DSTEP):
                    dvec = (lanes + (t * DSTEP + dd)) & (D - 1)
                    sv = plsc.load_gather(sref, [eids, dvec])
                    dv = plsc.load_gather(dref, [eids, dvec])
                    acc = acc + sv * dv
                return acc

            acc = lax.fori_loop(0, D // DSTEP, dstep,
                                jnp.zeros((L,), jnp.float32))
            obufs[b][pl.ds(g * L, L)] = 1.0 / (1.0 + jnp.exp(-acc))

    def step(c, b):
        @pl.when(c + 2 < NCHUNK)
        def _():
            wait_idx(c + 2, (b + 2) % NB)
            issue_rows((b + 2) % NB)
        wait_rows(b)

        @pl.when(c + 4 < NCHUNK)
        def _():
            issue_idx(c + 4, b)

        @pl.when(c >= NB)
        def _():
            wait_ostore(c - NB, b)
        compute(b)
        issue_ostore(c, b)

    issue_idx(0, 0)
    issue_idx(1, 1)
    issue_idx(2, 2)
    issue_idx(3, 3)
    wait_idx(0, 0)
    issue_rows(0)
    wait_idx(1, 1)
    issue_rows(1)

    def chunk_quad(i, carry):
        for b in range(NB):
            step(NB * i + b, b)
        return carry

    lax.fori_loop(0, NCHUNK // NB, chunk_quad, 0)
    step(NCHUNK - 1, 0)
    wait_ostore(NCHUNK - 4, 1)
    wait_ostore(NCHUNK - 3, 2)
    wait_ostore(NCHUNK - 2, 3)
    wait_ostore(NCHUNK - 1, 0)


@jax.jit
def _run(z, src, dst):
    mesh = plsc.VectorSubcoreMesh(
        core_axis_name="c", subcore_axis_name="s",
        num_cores=NC, num_subcores=NS)
    return pl.kernel(
        _sc_body,
        out_type=jax.ShapeDtypeStruct((E,), jnp.float32),
        mesh=mesh,
        compiler_params=pltpu.CompilerParams(needs_layout_passes=False),
        scratch_types=(
            [pltpu.VMEM((K,), jnp.int32) for _ in range(2 * NB)] +
            [pltpu.VMEM((K, D), jnp.float32) for _ in range(2 * NB)] +
            [pltpu.VMEM((K,), jnp.float32) for _ in range(NB)] +
            [pltpu.SemaphoreType.DMA for _ in range(3 * NB)]
        ),
    )(z, src, dst)


def kernel(z, edge_index):
    ei = edge_index.astype(jnp.int32)
    return _run(z, ei[0], ei[1])
